# baseline (speedup 1.0000x reference)
import functools

import jax
import jax.numpy as jnp
from jax import lax
from jax.experimental import pallas as pl
from jax.experimental.pallas import tpu as pltpu
from jax.experimental.pallas import tpu_sc as plsc

N = 16384
C = 256
OUT_D = 3 * C
T = 20

_LANES = 16
_NUM_CORES = 2
_NUM_SUBCORES = 16
_NW = _NUM_CORES * _NUM_SUBCORES
_ROWS_PER_W = N // _NW
_R = 32
_CHUNKS = _ROWS_PER_W // _R
_NBUF = 4
_GROUPS = _R // _LANES


def _tracklet_body(obj_hbm, pe_hbm, id_hbm, ts_hbm, te_hbm, out_hbm,
                   table_v, out_v, idx_v, *sems):
    sem_in = sems[:_NBUF]
    sem_out = sems[_NBUF:]
    wid = lax.axis_index("s") * _NUM_CORES + lax.axis_index("c")
    row0 = wid * _ROWS_PER_W

    pltpu.sync_copy(te_hbm, table_v)
    pltpu.sync_copy(ts_hbm.at[pl.ds(row0, _ROWS_PER_W)], idx_v)

    strips = ((obj_hbm, 0), (pe_hbm, C), (id_hbm, 2 * C))

    def issue_in(base, b):
        for src, c0 in strips:
            pltpu.async_copy(src.at[pl.ds(base, _R)],
                             out_v.at[b, :, pl.ds(c0, C)], sem_in[b])

    def wait_in(base, b):
        for src, c0 in strips:
            pltpu.make_async_copy(src.at[pl.ds(base, _R)],
                                  out_v.at[b, :, pl.ds(c0, C)],
                                  sem_in[b]).wait()

    lanes = lax.iota(jnp.int32, _LANES)
    zeros = jnp.zeros((_LANES,), jnp.int32)

    for ci in range(_NBUF - 1):
        issue_in(row0 + ci * _R, ci)

    def cycle_body(cg, _):
        for s in range(_NBUF):
            ci = cg * _NBUF + s
            base = row0 + ci * _R
            wait_in(base, s)
            ps = (s - 1) % _NBUF

            @pl.when(ci >= 1)
            def _wait_prev_out(ps=ps, base=base):
                pltpu.make_async_copy(
                    out_v.at[ps], out_hbm.at[pl.ds(base - _R, _R)],
                    sem_out[ps]).wait()

            @pl.when(ci + _NBUF - 1 < _CHUNKS)
            def _issue_next_in(ps=ps, base=base):
                issue_in(base + (_NBUF - 1) * _R, ps)

            @plsc.parallel_loop(0, _R)
            def _row_loop(i, ci=ci, s=s):
                dt_vec = idx_v[pl.ds(ci * _R + (i & -_LANES), _LANES)]
                dt = jnp.sum(jnp.where(lanes == (i & (_LANES - 1)), dt_vec,
                                       zeros))
                cols = [pl.ds(j * _LANES, _LANES)
                        for j in range(OUT_D // _LANES)]
                te = [table_v[dt, col] for col in cols]
                for col, te_col in zip(cols, te):
                    plsc.addupdate(out_v.at[s, i, col], te_col)

            pltpu.async_copy(out_v.at[s], out_hbm.at[pl.ds(base, _R)],
                             sem_out[s])
        return _

    lax.fori_loop(0, _CHUNKS // _NBUF, cycle_body, 0)
    pltpu.make_async_copy(
        out_v.at[_NBUF - 1],
        out_hbm.at[pl.ds(row0 + (_CHUNKS - 1) * _R, _R)],
        sem_out[_NBUF - 1]).wait()


@jax.jit
def kernel(obj_embedding, pe_3d, id_embedding, rel_timestep, temporal_embed):
    mesh = plsc.VectorSubcoreMesh(core_axis_name="c", subcore_axis_name="s")
    run = functools.partial(
        pl.kernel,
        mesh=mesh,
        compiler_params=pltpu.CompilerParams(needs_layout_passes=False),
        out_type=jax.ShapeDtypeStruct((N, OUT_D), jnp.float32),
        scratch_types=(
            [
                pltpu.VMEM((T, OUT_D), jnp.float32),
                pltpu.VMEM((_NBUF, _R, OUT_D), jnp.float32),
                pltpu.VMEM((_ROWS_PER_W,), jnp.int32),
            ]
            + [pltpu.SemaphoreType.DMA] * (2 * _NBUF)
        ),
    )(_tracklet_body)
    return run(obj_embedding, pe_3d, id_embedding, rel_timestep,
               temporal_embed)


# speedup vs baseline: 1.1312x; 1.1312x over previous
import functools

import jax
import jax.numpy as jnp
from jax import lax
from jax.experimental import pallas as pl
from jax.experimental.pallas import tpu as pltpu
from jax.experimental.pallas import tpu_sc as plsc

N = 16384
C = 256
OUT_D = 3 * C
T = 20

_LANES = 16
_NUM_CORES = 2
_NUM_SUBCORES = 16
_NW = _NUM_CORES * _NUM_SUBCORES
_ROWS_PER_W = N // _NW
_R = 16
_CHUNKS = _ROWS_PER_W // _R
_NBUF = 8
_GROUPS = _R // _LANES


def _tracklet_body(obj_hbm, pe_hbm, id_hbm, ts_hbm, te_hbm, out_hbm,
                   table_v, out_v, idx_v, *sems):
    sem_in = sems[:_NBUF]
    sem_out = sems[_NBUF:]
    wid = lax.axis_index("s") * _NUM_CORES + lax.axis_index("c")
    row0 = wid * _ROWS_PER_W

    pltpu.sync_copy(te_hbm, table_v)
    pltpu.sync_copy(ts_hbm.at[pl.ds(row0, _ROWS_PER_W)], idx_v)

    strips = ((obj_hbm, 0), (pe_hbm, C), (id_hbm, 2 * C))

    def issue_in(base, b):
        for src, c0 in strips:
            pltpu.async_copy(src.at[pl.ds(base, _R)],
                             out_v.at[b, :, pl.ds(c0, C)], sem_in[b])

    def wait_in(base, b):
        for src, c0 in strips:
            pltpu.make_async_copy(src.at[pl.ds(base, _R)],
                                  out_v.at[b, :, pl.ds(c0, C)],
                                  sem_in[b]).wait()

    lanes = lax.iota(jnp.int32, _LANES)
    zeros = jnp.zeros((_LANES,), jnp.int32)

    for ci in range(_NBUF - 1):
        issue_in(row0 + ci * _R, ci)

    def cycle_body(cg, _):
        for s in range(_NBUF):
            ci = cg * _NBUF + s
            base = row0 + ci * _R
            wait_in(base, s)

            @plsc.parallel_loop(0, _R)
            def _row_loop(i, ci=ci, s=s):
                dt_vec = idx_v[pl.ds(ci * _R + (i & -_LANES), _LANES)]
                dt = jnp.sum(jnp.where(lanes == (i & (_LANES - 1)), dt_vec,
                                       zeros))
                cols = [pl.ds(j * _LANES, _LANES)
                        for j in range(OUT_D // _LANES)]
                te = [table_v[dt, col] for col in cols]
                for col, te_col in zip(cols, te):
                    plsc.addupdate(out_v.at[s, i, col], te_col)

            ps = (s - 1) % _NBUF

            @pl.when(ci >= 1)
            def _wait_prev_out(ps=ps, base=base):
                pltpu.make_async_copy(
                    out_v.at[ps], out_hbm.at[pl.ds(base - _R, _R)],
                    sem_out[ps]).wait()

            @pl.when(ci + _NBUF - 1 < _CHUNKS)
            def _issue_next_in(ps=ps, base=base):
                issue_in(base + (_NBUF - 1) * _R, ps)

            pltpu.async_copy(out_v.at[s], out_hbm.at[pl.ds(base, _R)],
                             sem_out[s])
        return _

    lax.fori_loop(0, _CHUNKS // _NBUF, cycle_body, 0)
    pltpu.make_async_copy(
        out_v.at[_NBUF - 1],
        out_hbm.at[pl.ds(row0 + (_CHUNKS - 1) * _R, _R)],
        sem_out[_NBUF - 1]).wait()


@jax.jit
def kernel(obj_embedding, pe_3d, id_embedding, rel_timestep, temporal_embed):
    mesh = plsc.VectorSubcoreMesh(core_axis_name="c", subcore_axis_name="s")
    run = functools.partial(
        pl.kernel,
        mesh=mesh,
        compiler_params=pltpu.CompilerParams(needs_layout_passes=False),
        out_type=jax.ShapeDtypeStruct((N, OUT_D), jnp.float32),
        scratch_types=(
            [
                pltpu.VMEM((T, OUT_D), jnp.float32),
                pltpu.VMEM((_NBUF, _R, OUT_D), jnp.float32),
                pltpu.VMEM((_ROWS_PER_W,), jnp.int32),
            ]
            + [pltpu.SemaphoreType.DMA] * (2 * _NBUF)
        ),
    )(_tracklet_body)
    return run(obj_embedding, pe_3d, id_embedding, rel_timestep,
               temporal_embed)


# speedup vs baseline: 1.2297x; 1.0871x over previous
import functools

import jax
import jax.numpy as jnp
from jax import lax
from jax.experimental import pallas as pl
from jax.experimental.pallas import tpu as pltpu
from jax.experimental.pallas import tpu_sc as plsc

N = 16384
C = 256
OUT_D = 3 * C
T = 20

_LANES = 16
_NUM_CORES = 2
_NUM_SUBCORES = 16
_NW = _NUM_CORES * _NUM_SUBCORES
_ROWS_PER_W = N // _NW
_R = 32
_CHUNKS = _ROWS_PER_W // _R
_NBUF = 4
_GROUPS = _R // _LANES


def _tracklet_body(obj_hbm, pe_hbm, id_hbm, ts_hbm, te_hbm, out_hbm,
                   table_v, out_v, idx_v, *sems):
    sem_in = sems[:_NBUF]
    sem_out = sems[_NBUF:]
    wid = lax.axis_index("s") * _NUM_CORES + lax.axis_index("c")
    row0 = wid * _ROWS_PER_W

    strips = ((obj_hbm, 0), (pe_hbm, C), (id_hbm, 2 * C))

    def issue_in(base, b):
        for src, c0 in strips:
            pltpu.async_copy(src.at[pl.ds(base, _R)],
                             out_v.at[b, :, pl.ds(c0, C)], sem_in[b])

    def wait_in(base, b):
        for src, c0 in strips:
            pltpu.make_async_copy(src.at[pl.ds(base, _R)],
                                  out_v.at[b, :, pl.ds(c0, C)],
                                  sem_in[b]).wait()

    lanes = lax.iota(jnp.int32, _LANES)
    zeros = jnp.zeros((_LANES,), jnp.int32)

    for ci in range(_NBUF - 1):
        issue_in(row0 + ci * _R, ci)
    pltpu.sync_copy(te_hbm, table_v)
    pltpu.sync_copy(ts_hbm.at[pl.ds(row0, _ROWS_PER_W)], idx_v)

    def cycle_body(cg, _):
        for s in range(_NBUF):
            ci = cg * _NBUF + s
            base = row0 + ci * _R
            wait_in(base, s)

            @plsc.parallel_loop(0, _R)
            def _row_loop(i, ci=ci, s=s):
                dt_vec = idx_v[pl.ds(ci * _R + (i & -_LANES), _LANES)]
                dt = jnp.sum(jnp.where(lanes == (i & (_LANES - 1)), dt_vec,
                                       zeros))
                cols = [pl.ds(j * _LANES, _LANES)
                        for j in range(OUT_D // _LANES)]
                te = [table_v[dt, col] for col in cols]
                for col, te_col in zip(cols, te):
                    plsc.addupdate(out_v.at[s, i, col], te_col)

            ps = (s - 1) % _NBUF

            @pl.when(ci >= 1)
            def _wait_prev_out(ps=ps, base=base):
                pltpu.make_async_copy(
                    out_v.at[ps], out_hbm.at[pl.ds(base - _R, _R)],
                    sem_out[ps]).wait()

            @pl.when(ci + _NBUF - 1 < _CHUNKS)
            def _issue_next_in(ps=ps, base=base):
                issue_in(base + (_NBUF - 1) * _R, ps)

            pltpu.async_copy(out_v.at[s], out_hbm.at[pl.ds(base, _R)],
                             sem_out[s])
        return _

    lax.fori_loop(0, _CHUNKS // _NBUF, cycle_body, 0)
    pltpu.make_async_copy(
        out_v.at[_NBUF - 1],
        out_hbm.at[pl.ds(row0 + (_CHUNKS - 1) * _R, _R)],
        sem_out[_NBUF - 1]).wait()


@jax.jit
def kernel(obj_embedding, pe_3d, id_embedding, rel_timestep, temporal_embed):
    mesh = plsc.VectorSubcoreMesh(core_axis_name="c", subcore_axis_name="s")
    run = functools.partial(
        pl.kernel,
        mesh=mesh,
        compiler_params=pltpu.CompilerParams(needs_layout_passes=False),
        out_type=jax.ShapeDtypeStruct((N, OUT_D), jnp.float32),
        scratch_types=(
            [
                pltpu.VMEM((T, OUT_D), jnp.float32),
                pltpu.VMEM((_NBUF, _R, OUT_D), jnp.float32),
                pltpu.VMEM((_ROWS_PER_W,), jnp.int32),
            ]
            + [pltpu.SemaphoreType.DMA] * (2 * _NBUF)
        ),
    )(_tracklet_body)
    return run(obj_embedding, pe_3d, id_embedding, rel_timestep,
               temporal_embed)
